# baseline (speedup 1.0000x reference)
import functools

import jax
import jax.numpy as jnp
from jax import lax
from jax.experimental import pallas as pl
from jax.experimental.pallas import tpu as pltpu
from jax.experimental.pallas import tpu_sc as plsc

N = 10000
D = 128
E = 320000
NC, NS = 2, 16
NW = NC * NS
NPAD = 10240
RB = NPAD // NS
CHUNK = 128
CH = 80
UNROLL = 8
EPT = CH * CHUNK
E_PAD = EPT * NW
TCB = 256
GRID = NPAD // TCB

_mesh = plsc.VectorSubcoreMesh(
    core_axis_name="c", subcore_axis_name="s", num_cores=NC, num_subcores=NS)


@functools.partial(
    pl.kernel,
    out_type=jax.ShapeDtypeStruct((NC, NPAD, D), jnp.float32),
    mesh=_mesh,
    scratch_types=[
        pltpu.VMEM_SHARED((NPAD, D), jnp.float32),
        pltpu.VMEM((CHUNK, D), jnp.float32),
        pltpu.VMEM((UNROLL, CHUNK), jnp.int32),
        pltpu.SemaphoreType.DMA,
    ],
)
def _sc_deg(dst_hbm, zero_hbm, ones_hbm, out_hbm, acc, ones_v, didx, sem):
    c = lax.axis_index("c")
    s = lax.axis_index("s")
    wid = c * NS + s
    pltpu.sync_copy(zero_hbm, acc.at[pl.ds(s * RB, RB)])
    pltpu.sync_copy(ones_hbm, ones_v)
    plsc.subcore_barrier()

    def body(g, carry):
        pltpu.sync_copy(dst_hbm.at[wid, pl.ds(g * UNROLL, UNROLL)], didx)
        descs = []
        for j in range(UNROLL):
            descs.append(
                pltpu.async_copy(ones_v, acc.at[didx.at[j]], sem, add=True))
        for d in descs:
            d.wait()
        return carry

    lax.fori_loop(0, CH // UNROLL, body, 0)
    plsc.subcore_barrier()
    pltpu.sync_copy(acc.at[pl.ds(s * RB, RB)], out_hbm.at[c, pl.ds(s * RB, RB)])


@functools.partial(
    pl.kernel,
    out_type=jax.ShapeDtypeStruct((NC, NPAD, D), jnp.float32),
    mesh=_mesh,
    scratch_types=[
        pltpu.VMEM_SHARED((NPAD, D), jnp.float32),
        pltpu.VMEM((CHUNK, D), jnp.float32),
        pltpu.VMEM((CHUNK,), jnp.int32),
        pltpu.VMEM((CHUNK,), jnp.int32),
    ],
)
def _sc_scatter(z_hbm, src_hbm, dst_hbm, zero_hbm, out_hbm,
                acc, rows_v, sidx_v, didx_v):
    c = lax.axis_index("c")
    s = lax.axis_index("s")
    wid = c * NS + s
    pltpu.sync_copy(zero_hbm, acc.at[pl.ds(s * RB, RB)])
    plsc.subcore_barrier()

    def body(i, carry):
        base = wid * EPT + i * CHUNK
        pltpu.sync_copy(src_hbm.at[pl.ds(base, CHUNK)], sidx_v)
        pltpu.sync_copy(dst_hbm.at[pl.ds(base, CHUNK)], didx_v)
        pltpu.sync_copy(z_hbm.at[sidx_v], rows_v)
        pltpu.sync_copy(rows_v, acc.at[didx_v], add=True)
        return carry

    lax.fori_loop(0, CH, body, 0)
    plsc.subcore_barrier()
    pltpu.sync_copy(acc.at[pl.ds(s * RB, RB)], out_hbm.at[c, pl.ds(s * RB, RB)])


def _dinv_block(degp):
    deg = degp[0, :, 0:1] + degp[1, :, 0:1] + 1.0
    return lax.rsqrt(deg)


def _row_mask(i):
    rows = i * TCB + lax.broadcasted_iota(jnp.int32, (TCB, 1), 0)
    return rows < N


def _tc_first(degp_ref, x_ref, w_ref, z_ref):
    i = pl.program_id(0)
    dinv = _dinv_block(degp_ref[...])
    xw = jnp.dot(x_ref[...], w_ref[...], preferred_element_type=jnp.float32)
    z_ref[...] = jnp.where(_row_mask(i), xw * dinv, 0.0)


def _tc_mid(degp_ref, p_ref, z_ref, b_ref, w_ref, z2_ref):
    i = pl.program_id(0)
    dinv = _dinv_block(degp_ref[...])
    ssum = p_ref[0] + p_ref[1] + z_ref[...]
    h = jnp.maximum(dinv * ssum + b_ref[...], 0.0)
    h = jnp.where(_row_mask(i), h, 0.0)
    z2_ref[...] = jnp.dot(h, w_ref[...], preferred_element_type=jnp.float32) * dinv


def _tc_last(degp_ref, p_ref, z_ref, b_ref, wfc_ref, bfc_ref, y_ref):
    i = pl.program_id(0)
    dinv = _dinv_block(degp_ref[...])
    ssum = p_ref[0] + p_ref[1] + z_ref[...]
    h = jnp.maximum(dinv * ssum + b_ref[...], 0.0)
    h = jnp.where(_row_mask(i), h, 0.0)
    y_ref[...] = jnp.sum(h * wfc_ref[...], axis=1, keepdims=True) + bfc_ref[0, 0]


_degp_spec = pl.BlockSpec((NC, TCB, D), lambda i: (0, i, 0))
_rows_spec = pl.BlockSpec((TCB, D), lambda i: (i, 0))
_parts_spec = pl.BlockSpec((NC, TCB, D), lambda i: (0, i, 0))
_w_spec = pl.BlockSpec((D, D), lambda i: (0, 0))
_b_spec = pl.BlockSpec((1, D), lambda i: (0, 0))


def _tc_first_call(degp, x, w):
    return pl.pallas_call(
        _tc_first,
        grid=(GRID,),
        in_specs=[_degp_spec, _rows_spec, _w_spec],
        out_specs=_rows_spec,
        out_shape=jax.ShapeDtypeStruct((NPAD, D), jnp.float32),
    )(degp, x, w)


def _tc_mid_call(degp, parts, z, b2d, w):
    return pl.pallas_call(
        _tc_mid,
        grid=(GRID,),
        in_specs=[_degp_spec, _parts_spec, _rows_spec, _b_spec, _w_spec],
        out_specs=_rows_spec,
        out_shape=jax.ShapeDtypeStruct((NPAD, D), jnp.float32),
    )(degp, parts, z, b2d, w)


def _tc_last_call(degp, parts, z, b2d, wfc_row, bfc2d):
    return pl.pallas_call(
        _tc_last,
        grid=(GRID,),
        in_specs=[_degp_spec, _parts_spec, _rows_spec, _b_spec, _b_spec,
                  pl.BlockSpec((1, 1), lambda i: (0, 0))],
        out_specs=pl.BlockSpec((TCB, 1), lambda i: (i, 0)),
        out_shape=jax.ShapeDtypeStruct((NPAD, 1), jnp.float32),
    )(degp, parts, z, b2d, wfc_row, bfc2d)


def kernel(x, edge_index, W1, b1, W2, b2, Wfc, bfc):
    ei = edge_index.astype(jnp.int32)
    pad = jnp.full((E_PAD - E,), N, jnp.int32)
    srcp = jnp.concatenate([ei[0], pad])
    dstp = jnp.concatenate([ei[1], pad])
    dstp3 = dstp.reshape(NW, CH, CHUNK)

    zeroD = jnp.zeros((RB, D), jnp.float32)
    onesD = jnp.ones((CHUNK, D), jnp.float32)

    degp = _sc_deg(dstp3, zeroD, onesD)

    z1 = _tc_first_call(degp, x, W1)
    p1 = _sc_scatter(z1, srcp, dstp, zeroD)
    z2 = _tc_mid_call(degp, p1, z1, b1.reshape(1, D), W2)
    p2 = _sc_scatter(z2, srcp, dstp, zeroD)
    y = _tc_last_call(degp, p2, z2, b2.reshape(1, D),
                      Wfc.reshape(1, D), bfc.reshape(1, 1))
    return y[:N]


# speedup vs baseline: 1.4255x; 1.4255x over previous
import functools

import jax
import jax.numpy as jnp
from jax import lax
from jax.experimental import pallas as pl
from jax.experimental.pallas import tpu as pltpu
from jax.experimental.pallas import tpu_sc as plsc

N = 10000
D = 128
E = 320000
NC, NS = 2, 16
NW = NC * NS
NPAD = 10240
RB = NPAD // NS
CHUNK = 128
CH = 79
UNROLL = 8
EPT = CH * CHUNK
E_PAD = EPT * NW
TCB = 256
GRID = NPAD // TCB

_mesh = plsc.VectorSubcoreMesh(
    core_axis_name="c", subcore_axis_name="s", num_cores=NC, num_subcores=NS)


@functools.partial(
    pl.kernel,
    out_type=jax.ShapeDtypeStruct((NC, NPAD, D), jnp.float32),
    mesh=_mesh,
    scratch_types=[
        pltpu.VMEM_SHARED((NPAD, D), jnp.float32),
        pltpu.VMEM((CHUNK, D), jnp.float32),
        pltpu.VMEM((CHUNK,), jnp.int32),
    ],
)
def _sc_deg(dst_hbm, zero_hbm, ones_hbm, out_hbm, acc, ones_v, idx_v):
    c = lax.axis_index("c")
    s = lax.axis_index("s")
    wid = c * NS + s
    pltpu.sync_copy(zero_hbm, acc.at[pl.ds(s * RB, RB)])
    pltpu.sync_copy(ones_hbm, ones_v)
    plsc.subcore_barrier()

    def body(i, carry):
        base = wid * EPT + i * CHUNK
        pltpu.sync_copy(dst_hbm.at[pl.ds(base, CHUNK)], idx_v)
        pltpu.sync_copy(ones_v, acc.at[idx_v], add=True)
        return carry

    lax.fori_loop(0, CH, body, 0)
    plsc.subcore_barrier()
    pltpu.sync_copy(acc.at[pl.ds(s * RB, RB)], out_hbm.at[c, pl.ds(s * RB, RB)])


@functools.partial(
    pl.kernel,
    out_type=jax.ShapeDtypeStruct((NC, NPAD, D), jnp.float32),
    mesh=_mesh,
    scratch_types=[
        pltpu.VMEM_SHARED((NPAD, D), jnp.float32),
        pltpu.VMEM((CHUNK, D), jnp.float32),
        pltpu.VMEM((CHUNK,), jnp.int32),
        pltpu.VMEM((CHUNK,), jnp.int32),
    ],
)
def _sc_scatter(z_hbm, src_hbm, dst_hbm, zero_hbm, out_hbm,
                acc, rows_v, sidx_v, didx_v):
    c = lax.axis_index("c")
    s = lax.axis_index("s")
    wid = c * NS + s
    pltpu.sync_copy(zero_hbm, acc.at[pl.ds(s * RB, RB)])
    plsc.subcore_barrier()

    def body(i, carry):
        base = wid * EPT + i * CHUNK
        pltpu.sync_copy(src_hbm.at[pl.ds(base, CHUNK)], sidx_v)
        pltpu.sync_copy(dst_hbm.at[pl.ds(base, CHUNK)], didx_v)
        pltpu.sync_copy(z_hbm.at[sidx_v], rows_v)
        pltpu.sync_copy(rows_v, acc.at[didx_v], add=True)
        return carry

    lax.fori_loop(0, CH, body, 0)
    plsc.subcore_barrier()
    pltpu.sync_copy(acc.at[pl.ds(s * RB, RB)], out_hbm.at[c, pl.ds(s * RB, RB)])


def _dinv_block(degp):
    deg = degp[0, :, 0:1] + degp[1, :, 0:1] + 1.0
    return lax.rsqrt(deg)


def _row_mask(i):
    rows = i * TCB + lax.broadcasted_iota(jnp.int32, (TCB, 1), 0)
    return rows < N


def _tc_first(degp_ref, x_ref, w_ref, z_ref):
    i = pl.program_id(0)
    dinv = _dinv_block(degp_ref[...])
    xw = jnp.dot(x_ref[...], w_ref[...], preferred_element_type=jnp.float32)
    z_ref[...] = jnp.where(_row_mask(i), xw * dinv, 0.0)


def _tc_mid(degp_ref, p_ref, z_ref, b_ref, w_ref, z2_ref):
    i = pl.program_id(0)
    dinv = _dinv_block(degp_ref[...])
    ssum = p_ref[0] + p_ref[1] + z_ref[...]
    h = jnp.maximum(dinv * ssum + b_ref[...], 0.0)
    h = jnp.where(_row_mask(i), h, 0.0)
    z2_ref[...] = jnp.dot(h, w_ref[...], preferred_element_type=jnp.float32) * dinv


def _tc_last(degp_ref, p_ref, z_ref, b_ref, wfc_ref, bfc_ref, y_ref):
    i = pl.program_id(0)
    dinv = _dinv_block(degp_ref[...])
    ssum = p_ref[0] + p_ref[1] + z_ref[...]
    h = jnp.maximum(dinv * ssum + b_ref[...], 0.0)
    h = jnp.where(_row_mask(i), h, 0.0)
    y_ref[...] = jnp.sum(h * wfc_ref[...], axis=1, keepdims=True) + bfc_ref[0, 0]


_degp_spec = pl.BlockSpec((NC, TCB, D), lambda i: (0, i, 0))
_rows_spec = pl.BlockSpec((TCB, D), lambda i: (i, 0))
_parts_spec = pl.BlockSpec((NC, TCB, D), lambda i: (0, i, 0))
_w_spec = pl.BlockSpec((D, D), lambda i: (0, 0))
_b_spec = pl.BlockSpec((1, D), lambda i: (0, 0))


def _tc_first_call(degp, x, w):
    return pl.pallas_call(
        _tc_first,
        grid=(GRID,),
        in_specs=[_degp_spec, _rows_spec, _w_spec],
        out_specs=_rows_spec,
        out_shape=jax.ShapeDtypeStruct((NPAD, D), jnp.float32),
    )(degp, x, w)


def _tc_mid_call(degp, parts, z, b2d, w):
    return pl.pallas_call(
        _tc_mid,
        grid=(GRID,),
        in_specs=[_degp_spec, _parts_spec, _rows_spec, _b_spec, _w_spec],
        out_specs=_rows_spec,
        out_shape=jax.ShapeDtypeStruct((NPAD, D), jnp.float32),
    )(degp, parts, z, b2d, w)


def _tc_last_call(degp, parts, z, b2d, wfc_row, bfc2d):
    return pl.pallas_call(
        _tc_last,
        grid=(GRID,),
        in_specs=[_degp_spec, _parts_spec, _rows_spec, _b_spec, _b_spec,
                  pl.BlockSpec((1, 1), lambda i: (0, 0))],
        out_specs=pl.BlockSpec((TCB, 1), lambda i: (i, 0)),
        out_shape=jax.ShapeDtypeStruct((NPAD, 1), jnp.float32),
    )(degp, parts, z, b2d, wfc_row, bfc2d)


def kernel(x, edge_index, W1, b1, W2, b2, Wfc, bfc):
    ei = edge_index.astype(jnp.int32)
    pad = jnp.full((E_PAD - E,), N, jnp.int32)
    srcp = jnp.concatenate([ei[0], pad])
    dstp = jnp.concatenate([ei[1], pad])

    zeroD = jnp.zeros((RB, D), jnp.float32)
    onesD = jnp.ones((CHUNK, D), jnp.float32)

    degp = _sc_deg(dstp, zeroD, onesD)

    z1 = _tc_first_call(degp, x, W1)
    p1 = _sc_scatter(z1, srcp, dstp, zeroD)
    z2 = _tc_mid_call(degp, p1, z1, b1.reshape(1, D), W2)
    p2 = _sc_scatter(z2, srcp, dstp, zeroD)
    y = _tc_last_call(degp, p2, z2, b2.reshape(1, D),
                      Wfc.reshape(1, D), bfc.reshape(1, 1))
    return y[:N]
